# baseline (speedup 1.0000x reference)
import functools

import jax
import jax.numpy as jnp
from jax import lax
from jax.experimental import pallas as pl
from jax.experimental.pallas import tpu as pltpu
from jax.experimental.pallas import tpu_sc as plsc

_B = 4096
_H = 200
_E = 128
_MF = 128
_BLK = 64
_NW = 32
_CH = 128
_NCHUNK = 4
_BC = _B // _NCHUNK
_ROWS_W = _BC * _H // _NW
_NCH = _ROWS_W // _CH
_BW = _BC // _NW


def _sc_gather(src_iid, tgt_iid, src_uid, seq3d, iid2d, uid2d):
    mesh = plsc.VectorSubcoreMesh(core_axis_name="c", subcore_axis_name="s")

    @functools.partial(
        pl.kernel,
        out_type=(
            jax.ShapeDtypeStruct((_BC * _H, _E), jnp.float32),
            jax.ShapeDtypeStruct((_BC, _E), jnp.float32),
            jax.ShapeDtypeStruct((_BC, _E), jnp.float32),
        ),
        mesh=mesh,
        scratch_types=[
            pltpu.VMEM((_NCH, _CH), jnp.int32),
            pltpu.VMEM((_CH, _E), jnp.float32),
            pltpu.VMEM((_CH, _E), jnp.float32),
            pltpu.VMEM((_BW,), jnp.int32),
            pltpu.VMEM((_BW, _E), jnp.float32),
            pltpu.SemaphoreType.DMA,
            pltpu.SemaphoreType.DMA,
        ],
    )
    def k(src_iid_hbm, tgt_iid_hbm, src_uid_hbm, seq_hbm, iid_hbm, uid_hbm,
          ufea_hbm, v_hbm, u_hbm, idx_v, rows_a, rows_b, sidx, srows,
          sem_g, sem_s):
        wid = lax.axis_index("s") * 2 + lax.axis_index("c")
        base = wid * _ROWS_W

        pltpu.sync_copy(iid_hbm.at[wid], sidx)
        pltpu.async_copy(tgt_iid_hbm.at[sidx], srows, sem_g).wait()
        pltpu.sync_copy(srows, v_hbm.at[pl.ds(wid * _BW, _BW)])
        pltpu.sync_copy(uid_hbm.at[wid], sidx)
        pltpu.async_copy(src_uid_hbm.at[sidx], srows, sem_g).wait()
        pltpu.sync_copy(srows, u_hbm.at[pl.ds(wid * _BW, _BW)])

        pltpu.sync_copy(seq_hbm.at[wid], idx_v)
        pltpu.async_copy(src_iid_hbm.at[idx_v.at[0]], rows_a, sem_g)

        def step2(j2, _):
            j = j2 * 2
            pltpu.make_async_copy(src_iid_hbm.at[idx_v.at[0]], rows_a, sem_g).wait()
            pltpu.async_copy(src_iid_hbm.at[idx_v.at[j + 1]], rows_b, sem_g)
            pltpu.sync_copy(rows_a, ufea_hbm.at[pl.ds(base + j * _CH, _CH)])
            pltpu.make_async_copy(src_iid_hbm.at[idx_v.at[0]], rows_b, sem_g).wait()

            @pl.when(j2 + 1 < _NCH // 2)
            def _():
                pltpu.async_copy(src_iid_hbm.at[idx_v.at[j + 2]], rows_a, sem_g)

            pltpu.sync_copy(rows_b, ufea_hbm.at[pl.ds(base + (j + 1) * _CH, _CH)])
            return 0

        lax.fori_loop(0, _NCH // 2, step2, 0)

    return k(src_iid, tgt_iid, src_uid, seq3d, iid2d, uid2d)


def _tc_compute(ufea, mb3, u, v, w1, b1r, w2tile, dw1, db1r, wf, b2t,
                interpret=False):
    grid = (_BC // _BLK,)

    def body(uf_ref, mb_ref, u_ref, v_ref, w1_ref, b1_ref, w2_ref,
             dw1_ref, db1_ref, wf_ref, b2t_ref, out_ref):
        uf3 = uf_ref[...]
        uf2 = uf3.reshape(_BLK * _H, _E)
        h = jnp.maximum(
            jnp.dot(uf2, w1_ref[...], preferred_element_type=jnp.float32)
            + b1_ref[...], 0.0)
        ekr = jnp.dot(h, w2_ref[...], preferred_element_type=jnp.float32)
        ek3 = ekr.reshape(_BLK, _H, _E)
        t = ek3 - mb_ref[...]
        mx = jnp.max(t, axis=1, keepdims=True)
        p = jnp.exp(t - mx)
        att = p * (1.0 / jnp.sum(p, axis=1, keepdims=True))
        his = jnp.sum(att * uf3, axis=1)
        g = jnp.maximum(
            jnp.dot(his, dw1_ref[...], preferred_element_type=jnp.float32)
            + db1_ref[...], 0.0)
        P = jnp.dot(v_ref[...], wf_ref[...],
                    preferred_element_type=jnp.float32)
        P3 = P.reshape(_BLK, _MF, _E)
        s = jnp.sum(P3 * u_ref[...][:, None, :], axis=2)
        bv = jnp.dot(v_ref[...], b2t_ref[...],
                     preferred_element_type=jnp.float32)
        out_ref[...] = jnp.sum(g * s + u_ref[...] * bv, axis=1, keepdims=True)

    return pl.pallas_call(
        body,
        grid=grid,
        in_specs=[
            pl.BlockSpec((_BLK, _H, _E), lambda i: (i, 0, 0)),
            pl.BlockSpec((_BLK, _H, 1), lambda i: (i, 0, 0)),
            pl.BlockSpec((_BLK, _E), lambda i: (i, 0)),
            pl.BlockSpec((_BLK, _E), lambda i: (i, 0)),
            pl.BlockSpec((_E, _E), lambda i: (0, 0)),
            pl.BlockSpec((1, _E), lambda i: (0, 0)),
            pl.BlockSpec((_E, _E), lambda i: (0, 0)),
            pl.BlockSpec((_E, _MF), lambda i: (0, 0)),
            pl.BlockSpec((1, _MF), lambda i: (0, 0)),
            pl.BlockSpec((_E, _MF * _E), lambda i: (0, 0)),
            pl.BlockSpec((_E, _E), lambda i: (0, 0)),
        ],
        out_specs=pl.BlockSpec((_BLK, 1), lambda i: (i, 0)),
        out_shape=jax.ShapeDtypeStruct((_BC, 1), jnp.float32),
        interpret=interpret,
    )(ufea, mb3, u, v, w1, b1r, w2tile, dw1, db1r, wf, b2t)


def kernel(x, src_uid, src_iid, tgt_iid, ek_w1, ek_b1, ek_w2,
           dec_w1, dec_b1, dec_w2, dec_b2):
    x = x.astype(jnp.int32)
    b1r = ek_b1.reshape(1, _E)
    w2tile = jnp.tile(ek_w2, (1, _E))
    db1r = dec_b1.reshape(1, _MF)
    wf = jnp.transpose(dec_w2.reshape(_MF, _E, _E),
                       (2, 0, 1)).reshape(_E, _MF * _E)
    b2t = dec_b2.reshape(_E, _E).T
    mball = (x[:, 2:] == 0).astype(jnp.float32) * 1e8

    def gather_chunk(c):
        xc = x[c * _BC:(c + 1) * _BC]
        return _sc_gather(
            src_iid, tgt_iid, src_uid,
            xc[:, 2:].reshape(_NW, _NCH, _CH),
            xc[:, 1].reshape(_NW, _BW),
            xc[:, 0].reshape(_NW, _BW))

    outs = []
    gathered = gather_chunk(0)
    for c in range(_NCHUNK):
        nxt = gather_chunk(c + 1) if c + 1 < _NCHUNK else None
        ufea_rows, v, u = gathered
        out2 = _tc_compute(
            ufea_rows.reshape(_BC, _H, _E),
            mball[c * _BC:(c + 1) * _BC].reshape(_BC, _H, 1), u, v,
            ek_w1, b1r, w2tile, dec_w1, db1r, wf, b2t)
        outs.append(out2[:, 0])
        gathered = nxt
    return jnp.concatenate(outs, axis=0)


# speedup vs baseline: 1.0478x; 1.0478x over previous
import functools

import jax
import jax.numpy as jnp
from jax import lax
from jax.experimental import pallas as pl
from jax.experimental.pallas import tpu as pltpu
from jax.experimental.pallas import tpu_sc as plsc

_B = 4096
_H = 200
_E = 128
_MF = 128
_BLK = 64
_NW = 32
_CH = 128
_NCHUNK = 4
_BC = _B // _NCHUNK
_ROWS_W = _BC * _H // _NW
_NCH = _ROWS_W // _CH
_BW = _BC // _NW


def _sc_gather(src_iid, tgt_iid, src_uid, seq3d, iid2d, uid2d):
    mesh = plsc.VectorSubcoreMesh(core_axis_name="c", subcore_axis_name="s")

    @functools.partial(
        pl.kernel,
        out_type=(
            jax.ShapeDtypeStruct((_BC * _H, _E), jnp.float32),
            jax.ShapeDtypeStruct((_BC, _E), jnp.float32),
            jax.ShapeDtypeStruct((_BC, _E), jnp.float32),
        ),
        mesh=mesh,
        scratch_types=[
            pltpu.VMEM((_NCH, _CH), jnp.int32),
            pltpu.VMEM((_CH, _E), jnp.float32),
            pltpu.VMEM((_CH, _E), jnp.float32),
            pltpu.VMEM((_BW,), jnp.int32),
            pltpu.VMEM((_BW, _E), jnp.float32),
            pltpu.SemaphoreType.DMA,
            pltpu.SemaphoreType.DMA,
        ],
    )
    def k(src_iid_hbm, tgt_iid_hbm, src_uid_hbm, seq_hbm, iid_hbm, uid_hbm,
          ufea_hbm, v_hbm, u_hbm, idx_v, rows_a, rows_b, sidx, srows,
          sem_g, sem_s):
        wid = lax.axis_index("s") * 2 + lax.axis_index("c")
        base = wid * _ROWS_W

        pltpu.sync_copy(iid_hbm.at[wid], sidx)
        pltpu.async_copy(tgt_iid_hbm.at[sidx], srows, sem_g).wait()
        pltpu.sync_copy(srows, v_hbm.at[pl.ds(wid * _BW, _BW)])
        pltpu.sync_copy(uid_hbm.at[wid], sidx)
        pltpu.async_copy(src_uid_hbm.at[sidx], srows, sem_g).wait()
        pltpu.sync_copy(srows, u_hbm.at[pl.ds(wid * _BW, _BW)])

        pltpu.sync_copy(seq_hbm.at[wid], idx_v)
        pltpu.async_copy(src_iid_hbm.at[idx_v.at[0]], rows_a, sem_g)

        def step2(j2, _):
            j = j2 * 2
            pltpu.make_async_copy(src_iid_hbm.at[idx_v.at[0]], rows_a, sem_g).wait()
            pltpu.async_copy(src_iid_hbm.at[idx_v.at[j + 1]], rows_b, sem_g)
            pltpu.sync_copy(rows_a, ufea_hbm.at[pl.ds(base + j * _CH, _CH)])
            pltpu.make_async_copy(src_iid_hbm.at[idx_v.at[0]], rows_b, sem_g).wait()

            @pl.when(j2 + 1 < _NCH // 2)
            def _():
                pltpu.async_copy(src_iid_hbm.at[idx_v.at[j + 2]], rows_a, sem_g)

            pltpu.sync_copy(rows_b, ufea_hbm.at[pl.ds(base + (j + 1) * _CH, _CH)])
            return 0

        lax.fori_loop(0, _NCH // 2, step2, 0)

    return k(src_iid, tgt_iid, src_uid, seq3d, iid2d, uid2d)


def _tc_compute(ufea, mb3, u, v, w1, b1r, w2tile, dw1, db1r, wf, rmat, b2t,
                interpret=False):
    grid = (_BC // _BLK,)

    def body(uf_ref, mb_ref, u_ref, v_ref, w1_ref, b1_ref, w2_ref,
             dw1_ref, db1_ref, wf_ref, r_ref, b2t_ref, out_ref):
        uf3 = uf_ref[...]
        uf2 = uf3.reshape(_BLK * _H, _E)
        h = jnp.maximum(
            jnp.dot(uf2, w1_ref[...], preferred_element_type=jnp.float32)
            + b1_ref[...], 0.0)
        ekr = jnp.dot(h, w2_ref[...], preferred_element_type=jnp.float32)
        ek3 = ekr.reshape(_BLK, _H, _E)
        t = ek3 - mb_ref[...]
        mx = jnp.max(t, axis=1, keepdims=True)
        q = jnp.exp(t - mx)
        zden = jnp.sum(q, axis=1)
        his = jnp.sum(q * uf3, axis=1) / zden
        g = jnp.maximum(
            jnp.dot(his, dw1_ref[...], preferred_element_type=jnp.float32)
            + db1_ref[...], 0.0)
        P = jnp.dot(v_ref[...], wf_ref[...],
                    preferred_element_type=jnp.float32)
        urep = pltpu.repeat(u_ref[...], _MF, axis=1)
        s = jnp.dot(P * urep, r_ref[...],
                    preferred_element_type=jnp.float32)
        bv = jnp.dot(v_ref[...], b2t_ref[...],
                     preferred_element_type=jnp.float32)
        out_ref[...] = jnp.sum(g * s + u_ref[...] * bv, axis=1, keepdims=True)

    return pl.pallas_call(
        body,
        grid=grid,
        in_specs=[
            pl.BlockSpec((_BLK, _H, _E), lambda i: (i, 0, 0)),
            pl.BlockSpec((_BLK, _H, 1), lambda i: (i, 0, 0)),
            pl.BlockSpec((_BLK, _E), lambda i: (i, 0)),
            pl.BlockSpec((_BLK, _E), lambda i: (i, 0)),
            pl.BlockSpec((_E, _E), lambda i: (0, 0)),
            pl.BlockSpec((1, _E), lambda i: (0, 0)),
            pl.BlockSpec((_E, _E), lambda i: (0, 0)),
            pl.BlockSpec((_E, _MF), lambda i: (0, 0)),
            pl.BlockSpec((1, _MF), lambda i: (0, 0)),
            pl.BlockSpec((_E, _MF * _E), lambda i: (0, 0)),
            pl.BlockSpec((_MF * _E, _MF), lambda i: (0, 0)),
            pl.BlockSpec((_E, _E), lambda i: (0, 0)),
        ],
        out_specs=pl.BlockSpec((_BLK, 1), lambda i: (i, 0)),
        out_shape=jax.ShapeDtypeStruct((_BC, 1), jnp.float32),
        interpret=interpret,
    )(ufea, mb3, u, v, w1, b1r, w2tile, dw1, db1r, wf, rmat, b2t)


def kernel(x, src_uid, src_iid, tgt_iid, ek_w1, ek_b1, ek_w2,
           dec_w1, dec_b1, dec_w2, dec_b2):
    x = x.astype(jnp.int32)
    b1r = ek_b1.reshape(1, _E)
    w2tile = jnp.tile(ek_w2, (1, _E))
    db1r = dec_b1.reshape(1, _MF)
    wf = jnp.transpose(dec_w2.reshape(_MF, _E, _E),
                       (2, 0, 1)).reshape(_E, _MF * _E)
    b2t = dec_b2.reshape(_E, _E).T
    rmat = jnp.repeat(jnp.eye(_MF, dtype=jnp.float32), _E, axis=1).T
    mball = (x[:, 2:] == 0).astype(jnp.float32) * 1e8

    def gather_chunk(c):
        xc = x[c * _BC:(c + 1) * _BC]
        return _sc_gather(
            src_iid, tgt_iid, src_uid,
            xc[:, 2:].reshape(_NW, _NCH, _CH),
            xc[:, 1].reshape(_NW, _BW),
            xc[:, 0].reshape(_NW, _BW))

    outs = []
    gathered = gather_chunk(0)
    for c in range(_NCHUNK):
        nxt = gather_chunk(c + 1) if c + 1 < _NCHUNK else None
        ufea_rows, v, u = gathered
        out2 = _tc_compute(
            ufea_rows.reshape(_BC, _H, _E),
            mball[c * _BC:(c + 1) * _BC].reshape(_BC, _H, 1), u, v,
            ek_w1, b1r, w2tile, dec_w1, db1r, wf, rmat, b2t)
        outs.append(out2[:, 0])
        gathered = nxt
    return jnp.concatenate(outs, axis=0)


# speedup vs baseline: 1.0887x; 1.0390x over previous
import functools

import jax
import jax.numpy as jnp
from jax import lax
from jax.experimental import pallas as pl
from jax.experimental.pallas import tpu as pltpu
from jax.experimental.pallas import tpu_sc as plsc

_B = 4096
_H = 200
_E = 128
_MF = 128
_BLK = 64
_NW = 32
_CH = 128
_NCHUNK = 1
_BC = _B // _NCHUNK
_ROWS_W = _BC * _H // _NW
_NCH = _ROWS_W // _CH
_BW = _BC // _NW


def _sc_gather(src_iid, tgt_iid, src_uid, seq3d, iid2d, uid2d):
    mesh = plsc.VectorSubcoreMesh(core_axis_name="c", subcore_axis_name="s")

    @functools.partial(
        pl.kernel,
        out_type=(
            jax.ShapeDtypeStruct((_BC * _H, _E), jnp.float32),
            jax.ShapeDtypeStruct((_BC, _E), jnp.float32),
            jax.ShapeDtypeStruct((_BC, _E), jnp.float32),
        ),
        mesh=mesh,
        scratch_types=[
            pltpu.VMEM((_NCH, _CH), jnp.int32),
            pltpu.VMEM((_CH, _E), jnp.float32),
            pltpu.VMEM((_CH, _E), jnp.float32),
            pltpu.VMEM((_BW,), jnp.int32),
            pltpu.VMEM((_BW, _E), jnp.float32),
            pltpu.SemaphoreType.DMA,
            pltpu.SemaphoreType.DMA,
        ],
    )
    def k(src_iid_hbm, tgt_iid_hbm, src_uid_hbm, seq_hbm, iid_hbm, uid_hbm,
          ufea_hbm, v_hbm, u_hbm, idx_v, rows_a, rows_b, sidx, srows,
          sem_g, sem_s):
        wid = lax.axis_index("s") * 2 + lax.axis_index("c")
        base = wid * _ROWS_W

        pltpu.sync_copy(iid_hbm.at[wid], sidx)
        pltpu.async_copy(tgt_iid_hbm.at[sidx], srows, sem_g).wait()
        pltpu.sync_copy(srows, v_hbm.at[pl.ds(wid * _BW, _BW)])
        pltpu.sync_copy(uid_hbm.at[wid], sidx)
        pltpu.async_copy(src_uid_hbm.at[sidx], srows, sem_g).wait()
        pltpu.sync_copy(srows, u_hbm.at[pl.ds(wid * _BW, _BW)])

        pltpu.sync_copy(seq_hbm.at[wid], idx_v)
        pltpu.async_copy(src_iid_hbm.at[idx_v.at[0]], rows_a, sem_g)

        def step2(j2, _):
            j = j2 * 2
            pltpu.make_async_copy(src_iid_hbm.at[idx_v.at[0]], rows_a, sem_g).wait()
            pltpu.async_copy(src_iid_hbm.at[idx_v.at[j + 1]], rows_b, sem_g)
            pltpu.sync_copy(rows_a, ufea_hbm.at[pl.ds(base + j * _CH, _CH)])
            pltpu.make_async_copy(src_iid_hbm.at[idx_v.at[0]], rows_b, sem_g).wait()

            @pl.when(j2 + 1 < _NCH // 2)
            def _():
                pltpu.async_copy(src_iid_hbm.at[idx_v.at[j + 2]], rows_a, sem_g)

            pltpu.sync_copy(rows_b, ufea_hbm.at[pl.ds(base + (j + 1) * _CH, _CH)])
            return 0

        lax.fori_loop(0, _NCH // 2, step2, 0)

    return k(src_iid, tgt_iid, src_uid, seq3d, iid2d, uid2d)


def _tc_compute(ufea, mb3, u, v, w1, b1r, w2tile, dw1, db1r, wf, rmat, b2t,
                interpret=False):
    grid = (_BC // _BLK,)

    def body(uf_ref, mb_ref, u_ref, v_ref, w1_ref, b1_ref, w2_ref,
             dw1_ref, db1_ref, wf_ref, r_ref, b2t_ref, out_ref):
        uf3 = uf_ref[...]
        uf2 = uf3.reshape(_BLK * _H, _E)
        h = jnp.maximum(
            jnp.dot(uf2, w1_ref[...], preferred_element_type=jnp.float32)
            + b1_ref[...], 0.0)
        ekr = jnp.dot(h, w2_ref[...], preferred_element_type=jnp.float32)
        ek3 = ekr.reshape(_BLK, _H, _E)
        t = ek3 - mb_ref[...]
        mx = jnp.max(t, axis=1, keepdims=True)
        q = jnp.exp(t - mx)
        zden = jnp.sum(q, axis=1)
        his = jnp.sum(q * uf3, axis=1) / zden
        g = jnp.maximum(
            jnp.dot(his, dw1_ref[...], preferred_element_type=jnp.float32)
            + db1_ref[...], 0.0)
        P = jnp.dot(v_ref[...], wf_ref[...],
                    preferred_element_type=jnp.float32)
        urep = pltpu.repeat(u_ref[...], _MF, axis=1)
        s = jnp.dot(P * urep, r_ref[...],
                    preferred_element_type=jnp.float32)
        bv = jnp.dot(v_ref[...], b2t_ref[...],
                     preferred_element_type=jnp.float32)
        out_ref[...] = jnp.sum(g * s + u_ref[...] * bv, axis=1, keepdims=True)

    return pl.pallas_call(
        body,
        grid=grid,
        in_specs=[
            pl.BlockSpec((_BLK, _H, _E), lambda i: (i, 0, 0)),
            pl.BlockSpec((_BLK, _H, 1), lambda i: (i, 0, 0)),
            pl.BlockSpec((_BLK, _E), lambda i: (i, 0)),
            pl.BlockSpec((_BLK, _E), lambda i: (i, 0)),
            pl.BlockSpec((_E, _E), lambda i: (0, 0)),
            pl.BlockSpec((1, _E), lambda i: (0, 0)),
            pl.BlockSpec((_E, _E), lambda i: (0, 0)),
            pl.BlockSpec((_E, _MF), lambda i: (0, 0)),
            pl.BlockSpec((1, _MF), lambda i: (0, 0)),
            pl.BlockSpec((_E, _MF * _E), lambda i: (0, 0)),
            pl.BlockSpec((_MF * _E, _MF), lambda i: (0, 0)),
            pl.BlockSpec((_E, _E), lambda i: (0, 0)),
        ],
        out_specs=pl.BlockSpec((_BLK, 1), lambda i: (i, 0)),
        out_shape=jax.ShapeDtypeStruct((_BC, 1), jnp.float32),
        interpret=interpret,
    )(ufea, mb3, u, v, w1, b1r, w2tile, dw1, db1r, wf, rmat, b2t)


def kernel(x, src_uid, src_iid, tgt_iid, ek_w1, ek_b1, ek_w2,
           dec_w1, dec_b1, dec_w2, dec_b2):
    x = x.astype(jnp.int32)
    b1r = ek_b1.reshape(1, _E)
    w2tile = jnp.tile(ek_w2, (1, _E))
    db1r = dec_b1.reshape(1, _MF)
    wf = jnp.transpose(dec_w2.reshape(_MF, _E, _E),
                       (2, 0, 1)).reshape(_E, _MF * _E)
    b2t = dec_b2.reshape(_E, _E).T
    rmat = jnp.repeat(jnp.eye(_MF, dtype=jnp.float32), _E, axis=1).T
    mball = (x[:, 2:] == 0).astype(jnp.float32) * 1e8

    def gather_chunk(c):
        xc = x[c * _BC:(c + 1) * _BC]
        return _sc_gather(
            src_iid, tgt_iid, src_uid,
            xc[:, 2:].reshape(_NW, _NCH, _CH),
            xc[:, 1].reshape(_NW, _BW),
            xc[:, 0].reshape(_NW, _BW))

    outs = []
    gathered = gather_chunk(0)
    for c in range(_NCHUNK):
        nxt = gather_chunk(c + 1) if c + 1 < _NCHUNK else None
        ufea_rows, v, u = gathered
        out2 = _tc_compute(
            ufea_rows.reshape(_BC, _H, _E),
            mball[c * _BC:(c + 1) * _BC].reshape(_BC, _H, 1), u, v,
            ek_w1, b1r, w2tile, dec_w1, db1r, wf, rmat, b2t)
        outs.append(out2[:, 0])
        gathered = nxt
    return jnp.concatenate(outs, axis=0)
